# baseline (speedup 1.0000x reference)
import functools

import jax
import jax.numpy as jnp
from jax import lax
from jax.experimental import pallas as pl
from jax.experimental.pallas import tpu as pltpu
from jax.experimental.pallas import tpu_sc as plsc

F = 8
K = 16
L = 16
NW = 32
RCS = 2048
RT = 1024
NQ = 4


def _sc_lookup(x_hbm, evt_hbm, fo_hbm, fneg_hbm, xv0, xv1, fv0, fv1, evv, fov,
               sem_i0, sem_i1, sem_o0, sem_o1):
    wid = lax.axis_index("s") * 2 + lax.axis_index("c")
    n_rows = x_hbm.shape[0] // F
    rows_per = n_rows // NW
    base_row = wid * rows_per
    nc = rows_per // RCS

    pltpu.sync_copy(evt_hbm, evv)
    pltpu.sync_copy(fo_hbm, fov)

    lane = lax.iota(jnp.int32, L)
    lanefeat = (lane & 7) * K
    ev = [evv[pl.ds(k * L, L)] for k in range(K)]

    def in_copy(c, buf, sem):
        return pltpu.make_async_copy(
            x_hbm.at[pl.ds((base_row + c * RCS) * F, RCS * F)], buf, sem)

    def out_copy(c, buf, sem):
        return pltpu.make_async_copy(
            buf, fneg_hbm.at[pl.ds((base_row + c * RCS) * F, RCS * F)], sem)

    def compute_chunk(xv, fv):
        def group_body(g, _):
            xvv = xv[pl.ds(g * L, L)]
            vk = [(plsc.bitcast(xvv - ev[k], jnp.int32)
                   & jnp.int32(0x7FFFFFF0)) | k for k in range(K)]
            while len(vk) > 1:
                vk = [jnp.minimum(vk[2 * t], vk[2 * t + 1])
                      for t in range(len(vk) // 2)]
            idx = vk[0] & 15
            f = plsc.load_gather(fov, [lanefeat + idx])
            fv[pl.ds(g * L, L)] = -f
            return 0

        lax.fori_loop(0, RCS * F // L, group_body, 0, unroll=4)

    in_copy(0, xv0, sem_i0).start()

    def pair_body(i, _):
        c0 = i * 2
        in_copy(c0, xv0, sem_i0).wait()
        in_copy(c0 + 1, xv1, sem_i1).start()

        @pl.when(i > 0)
        def _():
            out_copy(c0 - 2, fv0, sem_o0).wait()
        compute_chunk(xv0, fv0)
        out_copy(c0, fv0, sem_o0).start()

        in_copy(c0 + 1, xv1, sem_i1).wait()

        @pl.when(i < nc // 2 - 1)
        def _():
            in_copy(c0 + 2, xv0, sem_i0).start()

        @pl.when(i > 0)
        def _():
            out_copy(c0 - 1, fv1, sem_o1).wait()
        compute_chunk(xv1, fv1)
        out_copy(c0 + 1, fv1, sem_o1).start()
        return 0

    lax.fori_loop(0, nc // 2, pair_body, 0, unroll=False)
    out_copy(nc - 2, fv0, sem_o0).wait()
    out_copy(nc - 1, fv1, sem_o1).wait()


def _run_sc_lookup(xf, evt, fof):
    mesh = plsc.VectorSubcoreMesh(core_axis_name="c", subcore_axis_name="s")
    return pl.kernel(
        _sc_lookup,
        mesh=mesh,
        out_type=jax.ShapeDtypeStruct(xf.shape, jnp.float32),
        scratch_types=[
            pltpu.VMEM((RCS * F,), jnp.float32),
            pltpu.VMEM((RCS * F,), jnp.float32),
            pltpu.VMEM((RCS * F,), jnp.float32),
            pltpu.VMEM((RCS * F,), jnp.float32),
            pltpu.VMEM((K * L,), jnp.float32),
            pltpu.VMEM((F * K,), jnp.float32),
            pltpu.SemaphoreType.DMA,
            pltpu.SemaphoreType.DMA,
            pltpu.SemaphoreType.DMA,
            pltpu.SemaphoreType.DMA,
        ],
        compiler_params=pltpu.CompilerParams(needs_layout_passes=False),
    )(xf, evt, fof)


def _tc_body(x_ref, f_ref, s_ref, ev_ref, m_ref, o_ref, *_unused):
    xb = x_ref[...]
    fb = f_ref[...]
    sel = s_ref[...]
    ev1 = ev_ref[0:1, :]
    xrep = jnp.dot(xb, sel, preferred_element_type=jnp.float32)
    frep = jnp.dot(fb, sel, preferred_element_type=jnp.float32)
    d = jnp.abs(xrep - ev1)
    e = jnp.exp(d * frep)
    ssum = jnp.dot(e, m_ref[...], preferred_element_type=jnp.float32)
    rb = jnp.dot(1.0 / ssum, sel, preferred_element_type=jnp.float32)
    o_ref[...] = e * rb


@jax.jit
def kernel(x, evaluate, focus):
    B, T, _ = x.shape
    N = B * T
    x2 = x.reshape(N, F)
    evt = jnp.tile(evaluate.T, (1, 2)).reshape(-1)
    fof = focus.reshape(-1)
    iota128 = jnp.arange(F * K, dtype=jnp.int32)
    sel = (iota128[None, :] // K == jnp.arange(F, dtype=jnp.int32)[:, None])
    sel = sel.astype(jnp.float32)
    msum = sel.T
    ev8 = jnp.tile(evaluate.reshape(1, F * K), (8, 1))

    nq_rows = N // NQ
    fnegs = [_run_sc_lookup(lax.dynamic_slice(x2, (q * nq_rows, 0),
                                              (nq_rows, F)).reshape(-1),
                            evt, fof)
             for q in range(NQ)]

    grid_q = nq_rows // RT
    out = None
    for q in range(NQ):
        row0 = q * grid_q
        fq = fnegs[q].reshape(nq_rows, F)
        if out is None:
            call = pl.pallas_call(
                _tc_body,
                grid=(grid_q,),
                in_specs=[
                    pl.BlockSpec((RT, F), lambda j, r0=row0: (r0 + j, 0)),
                    pl.BlockSpec((RT, F), lambda j: (j, 0)),
                    pl.BlockSpec((F, F * K), lambda j: (0, 0)),
                    pl.BlockSpec((F, F * K), lambda j: (0, 0)),
                    pl.BlockSpec((F * K, F), lambda j: (0, 0)),
                ],
                out_specs=pl.BlockSpec((RT, F * K), lambda j, r0=row0: (r0 + j, 0)),
                out_shape=jax.ShapeDtypeStruct((N, F * K), jnp.float32),
            )
            out = call(x2, fq, sel, ev8, msum)
        else:
            call = pl.pallas_call(
                lambda x_ref, f_ref, s_ref, ev_ref, m_ref, prev_ref, o_ref:
                    _tc_body(x_ref, f_ref, s_ref, ev_ref, m_ref, o_ref),
                grid=(grid_q,),
                in_specs=[
                    pl.BlockSpec((RT, F), lambda j, r0=row0: (r0 + j, 0)),
                    pl.BlockSpec((RT, F), lambda j: (j, 0)),
                    pl.BlockSpec((F, F * K), lambda j: (0, 0)),
                    pl.BlockSpec((F, F * K), lambda j: (0, 0)),
                    pl.BlockSpec((F * K, F), lambda j: (0, 0)),
                    pl.BlockSpec(memory_space=pl.ANY),
                ],
                out_specs=pl.BlockSpec((RT, F * K), lambda j, r0=row0: (r0 + j, 0)),
                out_shape=jax.ShapeDtypeStruct((N, F * K), jnp.float32),
                input_output_aliases={5: 0},
            )
            out = call(x2, fq, sel, ev8, msum, out)
    return out.reshape(B, T, F * K)


# speedup vs baseline: 1.4099x; 1.4099x over previous
import jax
import jax.numpy as jnp
from jax import lax
from jax.experimental import pallas as pl
from jax.experimental.pallas import tpu as pltpu
from jax.experimental.pallas import tpu_sc as plsc

F = 8
K = 16
L = 16
NW = 32
RC = 256
NEG_LOG2E = -1.4426950408889634


def _sc_kernel(x_hbm, evt_hbm, fo_hbm, out_hbm,
               xv0, xv1, ov0, ov1, evv, fov,
               sem_i0, sem_i1, sem_o0, sem_o1):
    wid = lax.axis_index("s") * 2 + lax.axis_index("c")
    n_rows = x_hbm.shape[0] // F
    rows_per = n_rows // NW
    base_row = wid * rows_per
    nc = rows_per // RC

    pltpu.sync_copy(evt_hbm, evv)
    pltpu.sync_copy(fo_hbm, fov)

    lane = lax.iota(jnp.int32, L)
    lanefeat = (lane & 7) * K
    obase = ((lane >> 3) << 7) + ((lane & 7) << 4)
    oidx = [obase | k for k in range(K)]
    ev = [evv[pl.ds(k * L, L)] for k in range(K)]

    def in_copy(c, buf, sem):
        return pltpu.make_async_copy(
            x_hbm.at[pl.ds((base_row + c * RC) * F, RC * F)], buf, sem)

    def out_copy(c, buf, sem):
        return pltpu.make_async_copy(
            buf, out_hbm.at[pl.ds((base_row + c * RC) * F * K, RC * F * K)], sem)

    def compute_chunk(xv, ov):
        def group_body(g, _):
            xvv = xv[pl.ds(g * L, L)]
            vk = [(plsc.bitcast(xvv - ev[k], jnp.int32)
                   & jnp.int32(0x7FFFFFF0)) | k for k in range(K)]
            m = vk
            while len(m) > 1:
                m = [jnp.minimum(m[2 * t], m[2 * t + 1])
                     for t in range(len(m) // 2)]
            idx = m[0] & 15
            f = plsc.load_gather(fov, [lanefeat + idx])
            c = -f
            d = [None] * K
            s = None
            for k in range(K):
                d[k] = jnp.exp(plsc.bitcast(vk[k], jnp.float32) * c)
                s = d[k] if s is None else s + d[k]
            r = 1.0 / s
            ovg = ov.at[pl.ds(g * 256, 256)]
            for k in range(K):
                plsc.store_scatter(ovg, [oidx[k]], d[k] * r)
            return 0

        lax.fori_loop(0, RC * F // L, group_body, 0, unroll=2)

    in_copy(0, xv0, sem_i0).start()

    def pair_body(i, _):
        c0 = i * 2
        in_copy(c0, xv0, sem_i0).wait()
        in_copy(c0 + 1, xv1, sem_i1).start()

        @pl.when(i > 0)
        def _():
            out_copy(c0 - 2, ov0, sem_o0).wait()
        compute_chunk(xv0, ov0)
        out_copy(c0, ov0, sem_o0).start()

        in_copy(c0 + 1, xv1, sem_i1).wait()

        @pl.when(i < nc // 2 - 1)
        def _():
            in_copy(c0 + 2, xv0, sem_i0).start()

        @pl.when(i > 0)
        def _():
            out_copy(c0 - 1, ov1, sem_o1).wait()
        compute_chunk(xv1, ov1)
        out_copy(c0 + 1, ov1, sem_o1).start()
        return 0

    lax.fori_loop(0, nc // 2, pair_body, 0, unroll=False)
    out_copy(nc - 2, ov0, sem_o0).wait()
    out_copy(nc - 1, ov1, sem_o1).wait()


@jax.jit
def kernel(x, evaluate, focus):
    B, T, _ = x.shape
    N = B * T
    evt = jnp.tile(evaluate.T, (1, 2)).reshape(-1)
    fof = focus.reshape(-1)
    mesh = plsc.VectorSubcoreMesh(core_axis_name="c", subcore_axis_name="s")
    run = pl.kernel(
        _sc_kernel,
        mesh=mesh,
        out_type=jax.ShapeDtypeStruct((N * F * K,), jnp.float32),
        scratch_types=[
            pltpu.VMEM((RC * F,), jnp.float32),
            pltpu.VMEM((RC * F,), jnp.float32),
            pltpu.VMEM((RC * F * K,), jnp.float32),
            pltpu.VMEM((RC * F * K,), jnp.float32),
            pltpu.VMEM((K * L,), jnp.float32),
            pltpu.VMEM((F * K,), jnp.float32),
            pltpu.SemaphoreType.DMA,
            pltpu.SemaphoreType.DMA,
            pltpu.SemaphoreType.DMA,
            pltpu.SemaphoreType.DMA,
        ],
        compiler_params=pltpu.CompilerParams(needs_layout_passes=False),
    )
    out = run(x.reshape(N * F), evt, fof)
    return out.reshape(B, T, F * K)


# speedup vs baseline: 1.7400x; 1.2342x over previous
import jax
import jax.numpy as jnp
from jax import lax
from jax.experimental import pallas as pl
from jax.experimental.pallas import tpu as pltpu
from jax.experimental.pallas import tpu_sc as plsc

F = 8
K = 16
L = 16
NW = 32
RC = 128
NEG_LOG2E = -1.4426950408889634


def _sc_kernel(x_hbm, evt_hbm, fo_hbm, out_hbm,
               xv0, xv1, ov0, ov1, evv, fov,
               sem_i0, sem_i1, sem_o0, sem_o1):
    wid = lax.axis_index("s") * 2 + lax.axis_index("c")
    n_rows = x_hbm.shape[0]
    rows_per = n_rows // NW
    base_row = wid * rows_per
    nc = rows_per // RC

    pltpu.sync_copy(evt_hbm, evv)
    pltpu.sync_copy(fo_hbm, fov)

    lane = lax.iota(jnp.int32, L)
    lanerow = lane >> 3
    lanecol = lane & 7
    lanefeat = (lane & 7) * K
    obase = ((lane >> 3) << 7) + ((lane & 7) << 4)
    oidx = [obase | k for k in range(K)]
    ev = [evv[pl.ds(k * L, L)] for k in range(K)]

    def in_copy(c, buf, sem):
        return pltpu.make_async_copy(
            x_hbm.at[pl.ds(base_row + c * RC, RC), :], buf, sem)

    def out_copy(c, buf, sem):
        return pltpu.make_async_copy(
            buf, out_hbm.at[pl.ds((base_row + c * RC) * F * K, RC * F * K)], sem)

    def compute_chunk(xv, ov):
        def group_body(g, _):
            xvv = plsc.load_gather(xv, [lanerow + g * 2, lanecol])
            vk = [(plsc.bitcast(xvv - ev[k], jnp.int32)
                   & jnp.int32(0x7FFFFFF0)) | k for k in range(K)]
            m = vk
            while len(m) > 1:
                m = [jnp.minimum(m[2 * t], m[2 * t + 1])
                     for t in range(len(m) // 2)]
            idx = m[0] & 15
            f = plsc.load_gather(fov, [lanefeat + idx])
            c = -f
            d = [None] * K
            s = None
            for k in range(K):
                d[k] = jnp.exp(plsc.bitcast(vk[k], jnp.float32) * c)
                s = d[k] if s is None else s + d[k]
            r = 1.0 / s
            ovg = ov.at[pl.ds(g * 256, 256)]
            for k in range(K):
                plsc.store_scatter(ovg, [oidx[k]], d[k] * r)
            return 0

        lax.fori_loop(0, RC * F // L, group_body, 0, unroll=2)

    in_copy(0, xv0, sem_i0).start()

    def pair_body(i, _):
        c0 = i * 2
        in_copy(c0, xv0, sem_i0).wait()
        in_copy(c0 + 1, xv1, sem_i1).start()

        @pl.when(i > 0)
        def _():
            out_copy(c0 - 2, ov0, sem_o0).wait()
        compute_chunk(xv0, ov0)
        out_copy(c0, ov0, sem_o0).start()

        in_copy(c0 + 1, xv1, sem_i1).wait()

        @pl.when(i < nc // 2 - 1)
        def _():
            in_copy(c0 + 2, xv0, sem_i0).start()

        @pl.when(i > 0)
        def _():
            out_copy(c0 - 1, ov1, sem_o1).wait()
        compute_chunk(xv1, ov1)
        out_copy(c0 + 1, ov1, sem_o1).start()
        return 0

    lax.fori_loop(0, nc // 2, pair_body, 0, unroll=False)
    out_copy(nc - 2, ov0, sem_o0).wait()
    out_copy(nc - 1, ov1, sem_o1).wait()


@jax.jit
def kernel(x, evaluate, focus):
    B, T, _ = x.shape
    N = B * T
    evt = jnp.tile(evaluate.T, (1, 2)).reshape(-1)
    fof = focus.reshape(-1)
    mesh = plsc.VectorSubcoreMesh(core_axis_name="c", subcore_axis_name="s")
    run = pl.kernel(
        _sc_kernel,
        mesh=mesh,
        out_type=jax.ShapeDtypeStruct((N * F * K,), jnp.float32),
        scratch_types=[
            pltpu.VMEM((RC, F), jnp.float32),
            pltpu.VMEM((RC, F), jnp.float32),
            pltpu.VMEM((RC * F * K,), jnp.float32),
            pltpu.VMEM((RC * F * K,), jnp.float32),
            pltpu.VMEM((K * L,), jnp.float32),
            pltpu.VMEM((F * K,), jnp.float32),
            pltpu.SemaphoreType.DMA,
            pltpu.SemaphoreType.DMA,
            pltpu.SemaphoreType.DMA,
            pltpu.SemaphoreType.DMA,
        ],
        compiler_params=pltpu.CompilerParams(needs_layout_passes=False),
    )
    out = run(x.reshape(N, F), evt, fof)
    return out.reshape(B, T, F * K)


# speedup vs baseline: 1.7666x; 1.0153x over previous
import jax
import jax.numpy as jnp
from jax import lax
from jax.experimental import pallas as pl
from jax.experimental.pallas import tpu as pltpu
from jax.experimental.pallas import tpu_sc as plsc

F = 8
K = 16
L = 16
NW = 32
RC = 128
NEG_LOG2E = -1.4426950408889634


def _sc_kernel(x_hbm, evt_hbm, fo_hbm, out_hbm,
               xv0, xv1, ov0, ov1, evv, fov,
               sem_i0, sem_i1, sem_o0, sem_o1):
    wid = lax.axis_index("s") * 2 + lax.axis_index("c")
    n_rows = x_hbm.shape[0]
    rows_per = n_rows // NW
    base_row = wid * rows_per
    nc = rows_per // RC

    pltpu.sync_copy(evt_hbm, evv)
    pltpu.sync_copy(fo_hbm, fov)

    lane = lax.iota(jnp.int32, L)
    lanerow = lane >> 3
    lanecol = lane & 7
    lanefeat = (lane & 7) * K
    obase = ((lane >> 3) << 7) + ((lane & 7) << 4)
    oidx = [obase | k for k in range(K)]
    ev = [evv[pl.ds(k * L, L)] for k in range(K)]

    def in_copy(c, buf, sem):
        return pltpu.make_async_copy(
            x_hbm.at[pl.ds(base_row + c * RC, RC), :], buf, sem)

    def out_copy(c, buf, sem):
        return pltpu.make_async_copy(
            buf, out_hbm.at[pl.ds((base_row + c * RC) * F * K, RC * F * K)], sem)

    def compute_chunk(xv, ov):
        def group_body(g, _):
            xvv = plsc.load_gather(xv, [lanerow + g * 2, lanecol])
            vk = [(plsc.bitcast(xvv - ev[k], jnp.int32)
                   & jnp.int32(0x7FFFFFF0)) | k for k in range(K)]
            m = vk
            while len(m) > 1:
                m = [jnp.minimum(m[2 * t], m[2 * t + 1])
                     for t in range(len(m) // 2)]
            idx = m[0] & 15
            f = plsc.load_gather(fov, [lanefeat + idx])
            c = -f
            d = [None] * K
            s = None
            for k in range(K):
                d[k] = jnp.exp(plsc.bitcast(vk[k], jnp.float32) * c)
                s = d[k] if s is None else s + d[k]
            r = 1.0 / s
            ovg = ov.at[pl.ds(g * 256, 256)]
            for k in range(K):
                plsc.store_scatter(ovg, [oidx[k]], d[k] * r)
            return 0

        lax.fori_loop(0, RC * F // L, group_body, 0, unroll=4)

    in_copy(0, xv0, sem_i0).start()

    def pair_body(i, _):
        c0 = i * 2
        in_copy(c0, xv0, sem_i0).wait()
        in_copy(c0 + 1, xv1, sem_i1).start()

        @pl.when(i > 0)
        def _():
            out_copy(c0 - 2, ov0, sem_o0).wait()
        compute_chunk(xv0, ov0)
        out_copy(c0, ov0, sem_o0).start()

        in_copy(c0 + 1, xv1, sem_i1).wait()

        @pl.when(i < nc // 2 - 1)
        def _():
            in_copy(c0 + 2, xv0, sem_i0).start()

        @pl.when(i > 0)
        def _():
            out_copy(c0 - 1, ov1, sem_o1).wait()
        compute_chunk(xv1, ov1)
        out_copy(c0 + 1, ov1, sem_o1).start()
        return 0

    lax.fori_loop(0, nc // 2, pair_body, 0, unroll=False)
    out_copy(nc - 2, ov0, sem_o0).wait()
    out_copy(nc - 1, ov1, sem_o1).wait()


@jax.jit
def kernel(x, evaluate, focus):
    B, T, _ = x.shape
    N = B * T
    evt = jnp.tile(evaluate.T, (1, 2)).reshape(-1)
    fof = focus.reshape(-1)
    mesh = plsc.VectorSubcoreMesh(core_axis_name="c", subcore_axis_name="s")
    run = pl.kernel(
        _sc_kernel,
        mesh=mesh,
        out_type=jax.ShapeDtypeStruct((N * F * K,), jnp.float32),
        scratch_types=[
            pltpu.VMEM((RC, F), jnp.float32),
            pltpu.VMEM((RC, F), jnp.float32),
            pltpu.VMEM((RC * F * K,), jnp.float32),
            pltpu.VMEM((RC * F * K,), jnp.float32),
            pltpu.VMEM((K * L,), jnp.float32),
            pltpu.VMEM((F * K,), jnp.float32),
            pltpu.SemaphoreType.DMA,
            pltpu.SemaphoreType.DMA,
            pltpu.SemaphoreType.DMA,
            pltpu.SemaphoreType.DMA,
        ],
        compiler_params=pltpu.CompilerParams(needs_layout_passes=False),
    )
    out = run(x.reshape(N, F), evt, fof)
    return out.reshape(B, T, F * K)


# speedup vs baseline: 1.7749x; 1.0047x over previous
import jax
import jax.numpy as jnp
from jax import lax
from jax.experimental import pallas as pl
from jax.experimental.pallas import tpu as pltpu
from jax.experimental.pallas import tpu_sc as plsc

F = 8
K = 16
L = 16
NW = 32
RC = 128
NEG_LOG2E = -1.4426950408889634


def _sc_kernel(x_hbm, evt_hbm, fo_hbm, out_hbm,
               xv0, xv1, ov0, ov1, evv, fov,
               sem_i0, sem_i1, sem_o0, sem_o1):
    wid = lax.axis_index("s") * 2 + lax.axis_index("c")
    n_rows = x_hbm.shape[0]
    rows_per = n_rows // NW
    base_row = wid * rows_per
    nc = rows_per // RC

    pltpu.sync_copy(evt_hbm, evv)
    pltpu.sync_copy(fo_hbm, fov)

    lane = lax.iota(jnp.int32, L)
    lanerow = lane >> 3
    lanecol = lane & 7
    lanefeat = (lane & 7) * K
    obase = ((lane >> 3) << 7) + ((lane & 7) << 4)
    oidx = [obase | k for k in range(K)]
    ev = [evv[pl.ds(k * L, L)] for k in range(K)]

    def in_copy(c, buf, sem):
        return pltpu.make_async_copy(
            x_hbm.at[pl.ds(base_row + c * RC, RC), :], buf, sem)

    def out_copy(c, buf, sem):
        return pltpu.make_async_copy(
            buf, out_hbm.at[pl.ds((base_row + c * RC) * F * K, RC * F * K)], sem)

    def compute_chunk(xv, ov):
        def group_body(g, _):
            xvv = plsc.load_gather(xv, [lanerow + g * 2, lanecol])
            vk = [(plsc.bitcast(xvv - ev[k], jnp.int32)
                   & jnp.int32(0x7FFFFFF0)) | k for k in range(K)]
            m = vk
            while len(m) > 1:
                m = [jnp.minimum(m[2 * t], m[2 * t + 1])
                     for t in range(len(m) // 2)]
            idx = m[0] & 15
            f = plsc.load_gather(fov, [lanefeat + idx])
            c = -f
            d = [None] * K
            s = None
            for k in range(K):
                d[k] = jnp.exp(plsc.bitcast(vk[k], jnp.float32) * c)
                s = d[k] if s is None else s + d[k]
            r = 1.0 / s
            ovg = ov.at[pl.ds(g * 256, 256)]
            for k in range(K):
                plsc.store_scatter(ovg, [oidx[k]], d[k] * r)
            return 0

        lax.fori_loop(0, RC * F // L, group_body, 0, unroll=8)

    in_copy(0, xv0, sem_i0).start()

    def pair_body(i, _):
        c0 = i * 2
        in_copy(c0, xv0, sem_i0).wait()
        in_copy(c0 + 1, xv1, sem_i1).start()

        @pl.when(i > 0)
        def _():
            out_copy(c0 - 2, ov0, sem_o0).wait()
        compute_chunk(xv0, ov0)
        out_copy(c0, ov0, sem_o0).start()

        in_copy(c0 + 1, xv1, sem_i1).wait()

        @pl.when(i < nc // 2 - 1)
        def _():
            in_copy(c0 + 2, xv0, sem_i0).start()

        @pl.when(i > 0)
        def _():
            out_copy(c0 - 1, ov1, sem_o1).wait()
        compute_chunk(xv1, ov1)
        out_copy(c0 + 1, ov1, sem_o1).start()
        return 0

    lax.fori_loop(0, nc // 2, pair_body, 0, unroll=False)
    out_copy(nc - 2, ov0, sem_o0).wait()
    out_copy(nc - 1, ov1, sem_o1).wait()


@jax.jit
def kernel(x, evaluate, focus):
    B, T, _ = x.shape
    N = B * T
    evt = jnp.tile(evaluate.T, (1, 2)).reshape(-1)
    fof = focus.reshape(-1)
    mesh = plsc.VectorSubcoreMesh(core_axis_name="c", subcore_axis_name="s")
    run = pl.kernel(
        _sc_kernel,
        mesh=mesh,
        out_type=jax.ShapeDtypeStruct((N * F * K,), jnp.float32),
        scratch_types=[
            pltpu.VMEM((RC, F), jnp.float32),
            pltpu.VMEM((RC, F), jnp.float32),
            pltpu.VMEM((RC * F * K,), jnp.float32),
            pltpu.VMEM((RC * F * K,), jnp.float32),
            pltpu.VMEM((K * L,), jnp.float32),
            pltpu.VMEM((F * K,), jnp.float32),
            pltpu.SemaphoreType.DMA,
            pltpu.SemaphoreType.DMA,
            pltpu.SemaphoreType.DMA,
            pltpu.SemaphoreType.DMA,
        ],
        compiler_params=pltpu.CompilerParams(needs_layout_passes=False),
    )
    out = run(x.reshape(N, F), evt, fof)
    return out.reshape(B, T, F * K)


# speedup vs baseline: 1.8165x; 1.0235x over previous
import jax
import jax.numpy as jnp
from jax import lax
from jax.experimental import pallas as pl
from jax.experimental.pallas import tpu as pltpu
from jax.experimental.pallas import tpu_sc as plsc

F = 8
K = 16
L = 16
NW = 32
RC = 128
NEG_LOG2E = -1.4426950408889634


def _sc_kernel(x_hbm, evt_hbm, fo_hbm, out_hbm,
               xv0, xv1, ov0, ov1, evv, fov,
               sem_i0, sem_i1, sem_o0, sem_o1):
    wid = lax.axis_index("s") * 2 + lax.axis_index("c")
    n_rows = x_hbm.shape[0]
    rows_per = n_rows // NW
    base_row = wid * rows_per
    nc = rows_per // RC

    pltpu.sync_copy(evt_hbm, evv)
    pltpu.sync_copy(fo_hbm, fov)

    lane = lax.iota(jnp.int32, L)
    lanerow = lane >> 3
    lanecol = lane & 7
    lanefeat = (lane & 7) * K
    obase = ((lane >> 3) << 7) + ((lane & 7) << 4)
    oidx = [obase | k for k in range(K)]
    ev = [evv[pl.ds(k * L, L)] for k in range(K)]

    def in_copy(c, buf, sem):
        return pltpu.make_async_copy(
            x_hbm.at[pl.ds(base_row + c * RC, RC), :], buf, sem)

    def out_copy(c, buf, sem):
        return pltpu.make_async_copy(
            buf, out_hbm.at[pl.ds((base_row + c * RC) * F * K, RC * F * K)], sem)

    def compute_chunk(xv, ov):
        def group_body(g, _):
            xvv = plsc.load_gather(xv, [lanerow + g * 2, lanecol])
            vk = [(plsc.bitcast(xvv - ev[k], jnp.int32)
                   & jnp.int32(0x7FFFFFF0)) | k for k in range(K)]
            m = vk
            while len(m) > 1:
                m = [jnp.minimum(m[2 * t], m[2 * t + 1])
                     for t in range(len(m) // 2)]
            idx = m[0] & 15
            c = plsc.load_gather(fov, [lanefeat + idx])
            d = [None] * K
            s = None
            for k in range(K):
                d[k] = jnp.exp(plsc.bitcast(vk[k], jnp.float32) * c)
                s = d[k] if s is None else s + d[k]
            r = 1.0 / s
            ovg = ov.at[pl.ds(g * 256, 256)]
            for k in range(K):
                plsc.store_scatter(ovg, [oidx[k]], d[k] * r)
            return 0

        lax.fori_loop(0, RC * F // L, group_body, 0, unroll=8)

    in_copy(0, xv0, sem_i0).start()

    def pair_body(i, _):
        c0 = i * 2
        in_copy(c0, xv0, sem_i0).wait()
        in_copy(c0 + 1, xv1, sem_i1).start()

        @pl.when(i > 0)
        def _():
            out_copy(c0 - 2, ov0, sem_o0).wait()
        compute_chunk(xv0, ov0)
        out_copy(c0, ov0, sem_o0).start()

        in_copy(c0 + 1, xv1, sem_i1).wait()

        @pl.when(i < nc // 2 - 1)
        def _():
            in_copy(c0 + 2, xv0, sem_i0).start()

        @pl.when(i > 0)
        def _():
            out_copy(c0 - 1, ov1, sem_o1).wait()
        compute_chunk(xv1, ov1)
        out_copy(c0 + 1, ov1, sem_o1).start()
        return 0

    lax.fori_loop(0, nc // 2, pair_body, 0, unroll=False)
    out_copy(nc - 2, ov0, sem_o0).wait()
    out_copy(nc - 1, ov1, sem_o1).wait()


@jax.jit
def kernel(x, evaluate, focus):
    B, T, _ = x.shape
    N = B * T
    evt = jnp.tile(evaluate.T, (1, 2)).reshape(-1)
    fof = -focus.reshape(-1)
    mesh = plsc.VectorSubcoreMesh(core_axis_name="c", subcore_axis_name="s")
    run = pl.kernel(
        _sc_kernel,
        mesh=mesh,
        out_type=jax.ShapeDtypeStruct((N * F * K,), jnp.float32),
        scratch_types=[
            pltpu.VMEM((RC, F), jnp.float32),
            pltpu.VMEM((RC, F), jnp.float32),
            pltpu.VMEM((RC * F * K,), jnp.float32),
            pltpu.VMEM((RC * F * K,), jnp.float32),
            pltpu.VMEM((K * L,), jnp.float32),
            pltpu.VMEM((F * K,), jnp.float32),
            pltpu.SemaphoreType.DMA,
            pltpu.SemaphoreType.DMA,
            pltpu.SemaphoreType.DMA,
            pltpu.SemaphoreType.DMA,
        ],
        compiler_params=pltpu.CompilerParams(needs_layout_passes=False),
    )
    out = run(x.reshape(N, F), evt, fof)
    return out.reshape(B, T, F * K)
